# baseline (speedup 1.0000x reference)
import functools

import jax
import jax.numpy as jnp
import numpy as np
from jax import lax
from jax.experimental import pallas as pl
from jax.experimental.pallas import tpu as pltpu
from jax.experimental.pallas import tpu_sc as plsc

N = 10000
NPAD = 10240
NFEAT = 128
NHID = 128
NCLASS = 16

NC = 2
NS = 16
L = 16
EQUANT = NC * NS * 512

_PERM = np.empty((NHID,), np.int32)
for _k in range(NHID // 32):
  for _m in range(16):
    _PERM[32 * _k + 2 * _m] = 32 * _k + _m
    _PERM[32 * _k + 2 * _m + 1] = 32 * _k + 16 + _m


def _spmm_sc(d: int, e_pad: int, bf16_dense: bool):
  chunk = 64 if d == 128 else 128
  n_phase = 2 if d == 128 else 1
  ept = e_pad // (NC * NS)
  n_chunks = ept // chunk
  nph = n_chunks // n_phase
  assert ept % chunk == 0 and nph % 4 == 0 and n_chunks % n_phase == 0
  rows_per_tile = NPAD // NS
  kd = d // L
  din = d // 2 if bf16_dense else d
  in_dtype = jnp.int32 if bf16_dense else jnp.float32

  mesh = plsc.VectorSubcoreMesh(
      core_axis_name="c", subcore_axis_name="s", num_cores=NC,
      num_subcores=NS)

  @functools.partial(
      pl.kernel,
      out_type=jax.ShapeDtypeStruct((NC, NPAD, d), jnp.float32),
      mesh=mesh,
      compiler_params=pltpu.CompilerParams(use_tc_tiling_on_sc=False),
      scratch_types=[
          pltpu.VMEM((nph, chunk), jnp.int32),
          pltpu.VMEM((nph, chunk), jnp.int32),
          pltpu.VMEM((nph, chunk), jnp.float32),
          pltpu.VMEM((4, chunk, din), in_dtype),
          pltpu.VMEM((2, chunk, d), jnp.float32),
          pltpu.VMEM_SHARED((NPAD, d), jnp.float32),
          [pltpu.SemaphoreType.DMA] * 4,
          [pltpu.SemaphoreType.DMA] * 2,
      ],
  )
  def spmm(row_h, col_h, w_h, dense_h, out_h, dst_v, src_v, w_v, rows_v,
           msg_v, acc_sh, gsems, ssems):
    c = lax.axis_index("c")
    s = lax.axis_index("s")
    cb = (c * NS + s) * n_chunks

    def gstart(i, b):
      pltpu.async_copy(dense_h.at[src_v.at[i]], rows_v.at[b], gsems[b])

    def gwait(i, b):
      pltpu.make_async_copy(
          dense_h.at[src_v.at[i]], rows_v.at[b], gsems[b]).wait()

    def sstart(i, b):
      pltpu.async_copy(msg_v.at[b], acc_sh.at[dst_v.at[i]], ssems[b],
                       add=True)

    def swait(i, b):
      pltpu.make_async_copy(msg_v.at[b], acc_sh.at[dst_v.at[i]],
                            ssems[b]).wait()

    zvec = jnp.zeros((L,), jnp.float32)

    def zbody(i, _):
      for k in range(kd):
        msg_v[0, i, pl.ds(k * L, L)] = zvec
      return 0

    lax.fori_loop(0, chunk, zbody, 0)
    for t in range(rows_per_tile // chunk):
      pltpu.sync_copy(
          msg_v.at[0], acc_sh.at[pl.ds(s * rows_per_tile + t * chunk, chunk)])
    plsc.subcore_barrier()

    for p in range(n_phase):
      pb = cb + p * nph
      pltpu.sync_copy(col_h.at[pl.ds(pb, nph)], src_v)
      pltpu.sync_copy(row_h.at[pl.ds(pb, nph)], dst_v)
      pltpu.sync_copy(w_h.at[pl.ds(pb, nph)], w_v)

      for b in range(3):
        gstart(b, b)

      def quad_body(q, _):
        for u in range(4):
          i = q * 4 + u
          gwait(i, u)

          @pl.when(i + 3 < nph)
          def _():
            gstart(i + 3, (u + 3) % 4)

          mb = u % 2

          def sbody(jj, _):
            w16 = w_v[i, pl.ds(jj * L, L)]
            for m in range(L):
              ws = w16[m]
              j = jj * L + m
              if bf16_dense:
                for k in range(kd // 2):
                  v = rows_v[u, j, pl.ds(L * k, L)]
                  lo = lax.bitcast_convert_type(v << 16, jnp.float32)
                  hi = lax.bitcast_convert_type(
                      v & jnp.int32(np.int32(-65536)), jnp.float32)
                  msg_v[mb, j, pl.ds(32 * k, L)] = lo * ws
                  msg_v[mb, j, pl.ds(32 * k + L, L)] = hi * ws
              else:
                for k in range(kd):
                  msg_v[mb, j, pl.ds(k * L, L)] = (
                      rows_v[u, j, pl.ds(k * L, L)] * ws)
            return 0

          lax.fori_loop(0, chunk // L, sbody, 0)

          sstart(i, mb)

          @pl.when(i >= 1)
          def _():
            swait(i - 1, 1 - mb)

        return 0

      lax.fori_loop(0, nph // 4, quad_body, 0)

      swait(nph - 1, (nph - 1) % 2)

    plsc.subcore_barrier()

    pltpu.sync_copy(
        acc_sh.at[pl.ds(s * rows_per_tile, rows_per_tile)],
        out_h.at[c, pl.ds(s * rows_per_tile, rows_per_tile)])

  return spmm


def _mm_bf16_body(x_ref, w_ref, o_ref):
  o_ref[...] = jnp.dot(x_ref[...], w_ref[...],
                       preferred_element_type=jnp.float32).astype(jnp.bfloat16)


def _combine1_body(p_ref, b_ref, w2_ref, pre_ref, h_ref, s2_ref):
  pre = p_ref[0] + p_ref[1] + b_ref[...]
  pre_ref[...] = pre
  hh = jnp.maximum(pre, 0.0)
  h_ref[...] = hh
  s2_ref[...] = jnp.dot(hh, w2_ref[...], preferred_element_type=jnp.float32)


def _combine2_body(p_ref, b_ref, pre_ref, out_ref):
  pre = p_ref[0] + p_ref[1] + b_ref[...]
  pre_ref[...] = pre
  m = jnp.max(pre, axis=1, keepdims=True)
  lse = jnp.log(jnp.sum(jnp.exp(pre - m), axis=1, keepdims=True)) + m
  out_ref[...] = pre - lse


_ROW_BLK = 2000


def kernel(x, edge_index, edge_weight, W1, b1, W2, b2):
  e = edge_index.shape[1]
  e_pad = ((e + EQUANT - 1) // EQUANT) * EQUANT
  pad = e_pad - e
  spread = jnp.arange(pad, dtype=jnp.int32) % jnp.int32(N)
  row = jnp.concatenate([edge_index[0], spread])
  col = jnp.concatenate([edge_index[1], spread])
  w = jnp.concatenate([edge_weight, jnp.zeros((pad,), jnp.float32)])

  grid = N // _ROW_BLK

  support1 = pl.pallas_call(
      _mm_bf16_body,
      out_shape=jax.ShapeDtypeStruct((N, NHID), jnp.bfloat16),
      grid=(grid,),
      in_specs=[
          pl.BlockSpec((_ROW_BLK, NFEAT), lambda i: (i, 0)),
          pl.BlockSpec((NFEAT, NHID), lambda i: (0, 0)),
      ],
      out_specs=pl.BlockSpec((_ROW_BLK, NHID), lambda i: (i, 0)),
  )(x, W1[:, _PERM])

  support1_packed = lax.bitcast_convert_type(
      support1.reshape(N, NHID // 2, 2), jnp.int32)
  c1 = 64
  p1 = _spmm_sc(NHID, e_pad, True)(
      row.reshape(e_pad // c1, c1), col.reshape(e_pad // c1, c1),
      w.reshape(e_pad // c1, c1), support1_packed)

  pre1, h, support2 = pl.pallas_call(
      _combine1_body,
      out_shape=(
          jax.ShapeDtypeStruct((N, NHID), jnp.float32),
          jax.ShapeDtypeStruct((N, NHID), jnp.float32),
          jax.ShapeDtypeStruct((N, NCLASS), jnp.float32),
      ),
      grid=(grid,),
      in_specs=[
          pl.BlockSpec((NC, _ROW_BLK, NHID), lambda i: (0, i, 0)),
          pl.BlockSpec((1, NHID), lambda i: (0, 0)),
          pl.BlockSpec((NHID, NCLASS), lambda i: (0, 0)),
      ],
      out_specs=(
          pl.BlockSpec((_ROW_BLK, NHID), lambda i: (i, 0)),
          pl.BlockSpec((_ROW_BLK, NHID), lambda i: (i, 0)),
          pl.BlockSpec((_ROW_BLK, NCLASS), lambda i: (i, 0)),
      ),
  )(p1, b1.reshape(1, NHID), W2)

  c2 = 128
  p2 = _spmm_sc(NCLASS, e_pad, False)(
      row.reshape(e_pad // c2, c2), col.reshape(e_pad // c2, c2),
      w.reshape(e_pad // c2, c2), support2)

  pre2, out = pl.pallas_call(
      _combine2_body,
      out_shape=(
          jax.ShapeDtypeStruct((N, NCLASS), jnp.float32),
          jax.ShapeDtypeStruct((N, NCLASS), jnp.float32),
      ),
      grid=(grid,),
      in_specs=[
          pl.BlockSpec((NC, _ROW_BLK, NCLASS), lambda i: (0, i, 0)),
          pl.BlockSpec((1, NCLASS), lambda i: (0, 0)),
      ],
      out_specs=(
          pl.BlockSpec((_ROW_BLK, NCLASS), lambda i: (i, 0)),
          pl.BlockSpec((_ROW_BLK, NCLASS), lambda i: (i, 0)),
      ),
  )(p2, b2.reshape(1, NCLASS))

  return (pre1, pre2, x, h, out)


# speedup vs baseline: 1.7678x; 1.7678x over previous
import functools

import jax
import jax.numpy as jnp
from jax import lax
from jax.experimental import pallas as pl
from jax.experimental.pallas import tpu as pltpu
from jax.experimental.pallas import tpu_sc as plsc

N = 10000
NPAD = 10240
NFEAT = 128
NHID = 128
NCLASS = 16

NC = 2
NS = 16
L = 16
EQUANT = NC * NS * 512


def _spmm_sc(d: int, e_pad: int):
  chunk = 64 if d == 128 else 128
  n_phase = 2 if d == 128 else 1
  ept = e_pad // (NC * NS)
  n_chunks = ept // chunk
  nph = n_chunks // n_phase
  assert ept % chunk == 0 and nph % 4 == 0 and n_chunks % n_phase == 0
  rows_per_tile = NPAD // NS
  kd = d // L

  mesh = plsc.VectorSubcoreMesh(
      core_axis_name="c", subcore_axis_name="s", num_cores=NC,
      num_subcores=NS)

  @functools.partial(
      pl.kernel,
      out_type=jax.ShapeDtypeStruct((NC, NPAD, d), jnp.float32),
      mesh=mesh,
      compiler_params=pltpu.CompilerParams(use_tc_tiling_on_sc=False),
      scratch_types=[
          pltpu.VMEM((nph, chunk), jnp.int32),
          pltpu.VMEM((nph, chunk), jnp.int32),
          pltpu.VMEM((nph, chunk), jnp.float32),
          pltpu.VMEM((4, chunk, d), jnp.float32),
          pltpu.VMEM_SHARED((NPAD, d), jnp.float32),
          [pltpu.SemaphoreType.DMA] * 4,
          [pltpu.SemaphoreType.DMA] * 4,
      ],
  )
  def spmm(row_h, col_h, w_h, dense_h, out_h, dst_v, src_v, w_v, rows_v,
           acc_sh, gsems, ssems):
    c = lax.axis_index("c")
    s = lax.axis_index("s")
    cb = (c * NS + s) * n_chunks

    def gstart(i, b):
      pltpu.async_copy(dense_h.at[src_v.at[i]], rows_v.at[b], gsems[b])

    def gwait(i, b):
      pltpu.make_async_copy(
          dense_h.at[src_v.at[i]], rows_v.at[b], gsems[b]).wait()

    def sstart(i, b):
      pltpu.async_copy(rows_v.at[b], acc_sh.at[dst_v.at[i]], ssems[b],
                       add=True)

    def swait(i, b):
      pltpu.make_async_copy(rows_v.at[b], acc_sh.at[dst_v.at[i]],
                            ssems[b]).wait()

    zvec = jnp.zeros((L,), jnp.float32)

    def zbody(i, _):
      for k in range(kd):
        rows_v[0, i, pl.ds(k * L, L)] = zvec
      return 0

    lax.fori_loop(0, chunk, zbody, 0)
    for t in range(rows_per_tile // chunk):
      pltpu.sync_copy(
          rows_v.at[0], acc_sh.at[pl.ds(s * rows_per_tile + t * chunk, chunk)])
    plsc.subcore_barrier()

    for p in range(n_phase):
      pb = cb + p * nph
      pltpu.sync_copy(col_h.at[pl.ds(pb, nph)], src_v)
      pltpu.sync_copy(row_h.at[pl.ds(pb, nph)], dst_v)
      pltpu.sync_copy(w_h.at[pl.ds(pb, nph)], w_v)

      for b in range(3):
        gstart(b, b)

      def quad_body(q, _):
        for u in range(4):
          i = q * 4 + u
          gwait(i, u)

          @plsc.parallel_loop(0, chunk // L, 1, unroll=2)
          def sbody(jj):
            w16 = w_v[i, pl.ds(jj * L, L)]
            for m in range(L):
              ws = w16[m]
              for k in range(kd):
                sl = (u, jj * L + m, pl.ds(k * L, L))
                rows_v[sl] = rows_v[sl] * ws

          sstart(i, u)

          nslot = (u + 3) % 4
          if u == 0:
            @pl.when(q == 0)
            def _():
              gstart(3, 3)

          @pl.when((i >= 1) & (i + 3 < nph))
          def _():
            swait(i - 1, nslot)
            gstart(i + 3, nslot)

        return 0

      lax.fori_loop(0, nph // 4, quad_body, 0)

      for j in range(nph - 4, nph):
        swait(j, j % 4)

    plsc.subcore_barrier()

    pltpu.sync_copy(
        acc_sh.at[pl.ds(s * rows_per_tile, rows_per_tile)],
        out_h.at[c, pl.ds(s * rows_per_tile, rows_per_tile)])

  return spmm


def _mm_body(x_ref, w_ref, o_ref):
  o_ref[...] = jnp.dot(x_ref[...], w_ref[...],
                       preferred_element_type=jnp.float32)


def _combine1_body(p_ref, b_ref, w2_ref, pre_ref, h_ref, s2_ref):
  pre = p_ref[0] + p_ref[1] + b_ref[...]
  pre_ref[...] = pre
  hh = jnp.maximum(pre, 0.0)
  h_ref[...] = hh
  s2_ref[...] = jnp.dot(hh, w2_ref[...], preferred_element_type=jnp.float32)


def _combine2_body(p_ref, b_ref, pre_ref, out_ref):
  pre = p_ref[0] + p_ref[1] + b_ref[...]
  pre_ref[...] = pre
  m = jnp.max(pre, axis=1, keepdims=True)
  lse = jnp.log(jnp.sum(jnp.exp(pre - m), axis=1, keepdims=True)) + m
  out_ref[...] = pre - lse


_ROW_BLK = 2000


def kernel(x, edge_index, edge_weight, W1, b1, W2, b2):
  e = edge_index.shape[1]
  e_pad = ((e + EQUANT - 1) // EQUANT) * EQUANT
  pad = e_pad - e
  spread = jnp.arange(pad, dtype=jnp.int32) % jnp.int32(N)
  row = jnp.concatenate([edge_index[0], spread])
  col = jnp.concatenate([edge_index[1], spread])
  w = jnp.concatenate([edge_weight, jnp.zeros((pad,), jnp.float32)])

  grid = N // _ROW_BLK

  support1 = pl.pallas_call(
      _mm_body,
      out_shape=jax.ShapeDtypeStruct((N, NHID), jnp.float32),
      grid=(grid,),
      in_specs=[
          pl.BlockSpec((_ROW_BLK, NFEAT), lambda i: (i, 0)),
          pl.BlockSpec((NFEAT, NHID), lambda i: (0, 0)),
      ],
      out_specs=pl.BlockSpec((_ROW_BLK, NHID), lambda i: (i, 0)),
  )(x, W1)

  c1 = 64
  p1 = _spmm_sc(NHID, e_pad)(
      row.reshape(e_pad // c1, c1), col.reshape(e_pad // c1, c1),
      w.reshape(e_pad // c1, c1), support1)

  pre1, h, support2 = pl.pallas_call(
      _combine1_body,
      out_shape=(
          jax.ShapeDtypeStruct((N, NHID), jnp.float32),
          jax.ShapeDtypeStruct((N, NHID), jnp.float32),
          jax.ShapeDtypeStruct((N, NCLASS), jnp.float32),
      ),
      grid=(grid,),
      in_specs=[
          pl.BlockSpec((NC, _ROW_BLK, NHID), lambda i: (0, i, 0)),
          pl.BlockSpec((1, NHID), lambda i: (0, 0)),
          pl.BlockSpec((NHID, NCLASS), lambda i: (0, 0)),
      ],
      out_specs=(
          pl.BlockSpec((_ROW_BLK, NHID), lambda i: (i, 0)),
          pl.BlockSpec((_ROW_BLK, NHID), lambda i: (i, 0)),
          pl.BlockSpec((_ROW_BLK, NCLASS), lambda i: (i, 0)),
      ),
  )(p1, b1.reshape(1, NHID), W2)

  c2 = 128
  p2 = _spmm_sc(NCLASS, e_pad)(
      row.reshape(e_pad // c2, c2), col.reshape(e_pad // c2, c2),
      w.reshape(e_pad // c2, c2), support2)

  pre2, out = pl.pallas_call(
      _combine2_body,
      out_shape=(
          jax.ShapeDtypeStruct((N, NCLASS), jnp.float32),
          jax.ShapeDtypeStruct((N, NCLASS), jnp.float32),
      ),
      grid=(grid,),
      in_specs=[
          pl.BlockSpec((NC, _ROW_BLK, NCLASS), lambda i: (0, i, 0)),
          pl.BlockSpec((1, NCLASS), lambda i: (0, 0)),
      ],
      out_specs=(
          pl.BlockSpec((_ROW_BLK, NCLASS), lambda i: (i, 0)),
          pl.BlockSpec((_ROW_BLK, NCLASS), lambda i: (i, 0)),
      ),
  )(p2, b2.reshape(1, NCLASS))

  return (pre1, pre2, x, h, out)


# speedup vs baseline: 1.7682x; 1.0002x over previous
import functools

import jax
import jax.numpy as jnp
from jax import lax
from jax.experimental import pallas as pl
from jax.experimental.pallas import tpu as pltpu
from jax.experimental.pallas import tpu_sc as plsc

N = 10000
NPAD = 10240
NFEAT = 128
NHID = 128
NCLASS = 16

NC = 2
NS = 16
L = 16
EQUANT = NC * NS * 512


def _spmm_sc(d: int, e_pad: int):
  chunk = 64 if d == 128 else 128
  n_phase = 2 if d == 128 else 1
  ept = e_pad // (NC * NS)
  n_chunks = ept // chunk
  nph = n_chunks // n_phase
  assert ept % chunk == 0 and nph % 4 == 0 and n_chunks % n_phase == 0
  rows_per_tile = NPAD // NS
  kd = d // L

  mesh = plsc.VectorSubcoreMesh(
      core_axis_name="c", subcore_axis_name="s", num_cores=NC,
      num_subcores=NS)

  @functools.partial(
      pl.kernel,
      out_type=jax.ShapeDtypeStruct((NC, NPAD, d), jnp.float32),
      mesh=mesh,
      compiler_params=pltpu.CompilerParams(use_tc_tiling_on_sc=False),
      scratch_types=[
          pltpu.VMEM((nph, chunk), jnp.int32),
          pltpu.VMEM((nph * chunk,), jnp.int32),
          pltpu.VMEM((nph * chunk,), jnp.float32),
          pltpu.VMEM((4, chunk, d), jnp.float32),
          pltpu.VMEM_SHARED((NPAD, d), jnp.float32),
          [pltpu.SemaphoreType.DMA] * 4,
          [pltpu.SemaphoreType.DMA] * 4,
      ],
  )
  def spmm(row_h, col_h, w_h, dense_h, out_h, dst_v, src_v, w_v, rows_v,
           acc_sh, gsems, ssems):
    c = lax.axis_index("c")
    s = lax.axis_index("s")
    cb = (c * NS + s) * n_chunks

    def gstart(i, b):
      pltpu.async_copy(
          dense_h.at[src_v.at[pl.ds(i * chunk, chunk)]], rows_v.at[b],
          gsems[b])

    def gwait(i, b):
      pltpu.make_async_copy(
          dense_h.at[src_v.at[pl.ds(i * chunk, chunk)]], rows_v.at[b],
          gsems[b]).wait()

    def sstart(i, b):
      pltpu.async_copy(rows_v.at[b], acc_sh.at[dst_v.at[i]], ssems[b],
                       add=True)

    def swait(i, b):
      pltpu.make_async_copy(rows_v.at[b], acc_sh.at[dst_v.at[i]],
                            ssems[b]).wait()

    zvec = jnp.zeros((L,), jnp.float32)

    def zbody(i, _):
      for k in range(kd):
        rows_v[0, i, pl.ds(k * L, L)] = zvec
      return 0

    lax.fori_loop(0, chunk, zbody, 0)
    for t in range(rows_per_tile // chunk):
      pltpu.sync_copy(
          rows_v.at[0], acc_sh.at[pl.ds(s * rows_per_tile + t * chunk, chunk)])
    plsc.subcore_barrier()

    for p in range(n_phase):
      pb = cb + p * nph
      pltpu.sync_copy(col_h.at[pl.ds(pb * chunk, nph * chunk)], src_v)
      pltpu.sync_copy(row_h.at[pl.ds(pb, nph)], dst_v)
      pltpu.sync_copy(w_h.at[pl.ds(pb * chunk, nph * chunk)], w_v)

      for b in range(3):
        gstart(b, b)

      def quad_body(q, _):
        for u in range(4):
          i = q * 4 + u
          gwait(i, u)

          @plsc.parallel_loop(0, chunk // L, 1, unroll=2)
          def sbody(jj):
            w16 = w_v[pl.ds(i * chunk + jj * L, L)]
            for m in range(L):
              ws = w16[m]
              for k in range(kd):
                sl = (u, jj * L + m, pl.ds(k * L, L))
                rows_v[sl] = rows_v[sl] * ws

          sstart(i, u)

          nslot = (u + 3) % 4
          if u == 0:
            @pl.when(q == 0)
            def _():
              gstart(3, 3)

          @pl.when((i >= 1) & (i + 3 < nph))
          def _():
            swait(i - 1, nslot)
            gstart(i + 3, nslot)

        return 0

      lax.fori_loop(0, nph // 4, quad_body, 0)

      for j in range(nph - 4, nph):
        swait(j, j % 4)

    plsc.subcore_barrier()

    pltpu.sync_copy(
        acc_sh.at[pl.ds(s * rows_per_tile, rows_per_tile)],
        out_h.at[c, pl.ds(s * rows_per_tile, rows_per_tile)])

  return spmm


def _mm_body(x_ref, w_ref, o_ref):
  o_ref[...] = jnp.dot(x_ref[...], w_ref[...],
                       preferred_element_type=jnp.float32)


def _combine1_body(p_ref, b_ref, w2_ref, pre_ref, h_ref, s2_ref):
  pre = p_ref[0] + p_ref[1] + b_ref[...]
  pre_ref[...] = pre
  hh = jnp.maximum(pre, 0.0)
  h_ref[...] = hh
  s2_ref[...] = jnp.dot(hh, w2_ref[...], preferred_element_type=jnp.float32)


def _combine2_body(p_ref, b_ref, pre_ref, out_ref):
  pre = p_ref[0] + p_ref[1] + b_ref[...]
  pre_ref[...] = pre
  m = jnp.max(pre, axis=1, keepdims=True)
  lse = jnp.log(jnp.sum(jnp.exp(pre - m), axis=1, keepdims=True)) + m
  out_ref[...] = pre - lse


_ROW_BLK = 2000


def kernel(x, edge_index, edge_weight, W1, b1, W2, b2):
  e = edge_index.shape[1]
  e_pad = ((e + EQUANT - 1) // EQUANT) * EQUANT
  pad = e_pad - e
  spread = jnp.arange(pad, dtype=jnp.int32) % jnp.int32(N)
  row = jnp.concatenate([edge_index[0], spread])
  col = jnp.concatenate([edge_index[1], spread])
  w = jnp.concatenate([edge_weight, jnp.zeros((pad,), jnp.float32)])

  grid = N // _ROW_BLK

  support1 = pl.pallas_call(
      _mm_body,
      out_shape=jax.ShapeDtypeStruct((N, NHID), jnp.float32),
      grid=(grid,),
      in_specs=[
          pl.BlockSpec((_ROW_BLK, NFEAT), lambda i: (i, 0)),
          pl.BlockSpec((NFEAT, NHID), lambda i: (0, 0)),
      ],
      out_specs=pl.BlockSpec((_ROW_BLK, NHID), lambda i: (i, 0)),
  )(x, W1)

  c1 = 64
  p1 = _spmm_sc(NHID, e_pad)(
      row.reshape(e_pad // c1, c1), col, w, support1)

  pre1, h, support2 = pl.pallas_call(
      _combine1_body,
      out_shape=(
          jax.ShapeDtypeStruct((N, NHID), jnp.float32),
          jax.ShapeDtypeStruct((N, NHID), jnp.float32),
          jax.ShapeDtypeStruct((N, NCLASS), jnp.float32),
      ),
      grid=(grid,),
      in_specs=[
          pl.BlockSpec((NC, _ROW_BLK, NHID), lambda i: (0, i, 0)),
          pl.BlockSpec((1, NHID), lambda i: (0, 0)),
          pl.BlockSpec((NHID, NCLASS), lambda i: (0, 0)),
      ],
      out_specs=(
          pl.BlockSpec((_ROW_BLK, NHID), lambda i: (i, 0)),
          pl.BlockSpec((_ROW_BLK, NHID), lambda i: (i, 0)),
          pl.BlockSpec((_ROW_BLK, NCLASS), lambda i: (i, 0)),
      ),
  )(p1, b1.reshape(1, NHID), W2)

  c2 = 128
  p2 = _spmm_sc(NCLASS, e_pad)(
      row.reshape(e_pad // c2, c2), col, w, support2)

  pre2, out = pl.pallas_call(
      _combine2_body,
      out_shape=(
          jax.ShapeDtypeStruct((N, NCLASS), jnp.float32),
          jax.ShapeDtypeStruct((N, NCLASS), jnp.float32),
      ),
      grid=(grid,),
      in_specs=[
          pl.BlockSpec((NC, _ROW_BLK, NCLASS), lambda i: (0, i, 0)),
          pl.BlockSpec((1, NCLASS), lambda i: (0, 0)),
      ],
      out_specs=(
          pl.BlockSpec((_ROW_BLK, NCLASS), lambda i: (i, 0)),
          pl.BlockSpec((_ROW_BLK, NCLASS), lambda i: (i, 0)),
      ),
  )(p2, b2.reshape(1, NCLASS))

  return (pre1, pre2, x, h, out)
